# Initial kernel scaffold; baseline (speedup 1.0000x reference)
import jax
import jax.numpy as jnp
from jax.experimental import pallas as pl


def kernel(x, edge_index, edge_weight, W_emb, b_emb, W1, b1, W2, b2, W3, b3, W4, b4, Wr1, br1, Wr2, br2, Wr3, br3):
    raise NotImplementedError("write your pallas kernel here")


# speedup vs baseline: 4.7389x; 4.7389x over previous
import functools

import jax
import jax.numpy as jnp
from jax import lax
from jax.experimental import pallas as pl
from jax.experimental.pallas import tpu as pltpu
from jax.experimental.pallas import tpu_sc as plsc

N = 10000
E = 160000
D = 256
HD = 128
NCORE = 2
NSUB = 16
LANES = 16
EPT = 10240
E_PAD = EPT * NSUB
ROWS = E_PAD // 128
CHUNK = 256
NCHUNK = EPT // CHUNK
RB = 1000
NB = N // RB


def _mesh():
    return plsc.VectorSubcoreMesh(core_axis_name="c", subcore_axis_name="s")


def _deg_body(dst2d, wflat, degp, dbuf, wbuf, zbuf, acc, sem):
    c = lax.axis_index("c")
    s = lax.axis_index("s")
    wid = c * NSUB + s
    row0 = wid * (ROWS // (NCORE * NSUB))
    ebase = wid * (E_PAD // (NCORE * NSUB))

    @pl.when(s == 0)
    def _():
        zbuf[...] = jnp.zeros((1024,), jnp.float32)
        for k in range(9):
            pltpu.sync_copy(zbuf, acc.at[pl.ds(k * 1024, 1024)])
        pltpu.sync_copy(zbuf.at[pl.ds(0, 784)], acc.at[pl.ds(9216, 784)])

    plsc.subcore_barrier()

    pltpu.sync_copy(dst2d.at[pl.ds(row0, 40)], dbuf)
    pltpu.sync_copy(wflat.at[pl.ds(ebase, 5120)], wbuf)
    cps = [
        pltpu.async_copy(wbuf.at[pl.ds(r * 128, 128)], acc.at[dbuf.at[r]],
                         sem, add=True)
        for r in range(40)
    ]
    for cp in cps:
        cp.wait()

    plsc.subcore_barrier()

    @pl.when(s == 0)
    def _():
        pltpu.sync_copy(acc.at[pl.ds(0, 5120)], wbuf)
        pltpu.sync_copy(wbuf, degp.at[pl.ds(c * N, 5120)])
        pltpu.sync_copy(acc.at[pl.ds(5120, 4880)], wbuf.at[pl.ds(0, 4880)])
        pltpu.sync_copy(wbuf.at[pl.ds(0, 4880)],
                        degp.at[pl.ds(c * N + 5120, 4880)])


def _deg_partials(dst2d, wflat):
    return pl.kernel(
        _deg_body,
        out_type=jax.ShapeDtypeStruct((NCORE * N,), jnp.float32),
        mesh=_mesh(),
        scratch_types=[
            pltpu.VMEM((40, 128), jnp.int32),
            pltpu.VMEM((5120,), jnp.float32),
            pltpu.VMEM((1024,), jnp.float32),
            pltpu.VMEM_SHARED((N,), jnp.float32),
            pltpu.SemaphoreType.DMA,
        ],
    )(dst2d, wflat)


def _agg_body(y, src2d, dst2d, wflat, agg, sidx, didx, wbuf, rows, acc,
              sem, sem2):
    c = lax.axis_index("c")
    s = lax.axis_index("s")
    cN = c * N

    rows[...] = jnp.zeros((CHUNK, HD), jnp.float32)
    t0 = s * 624
    pltpu.sync_copy(rows, acc.at[pl.ds(t0, CHUNK)])
    pltpu.sync_copy(rows, acc.at[pl.ds(t0 + CHUNK, CHUNK)])
    pltpu.sync_copy(rows.at[pl.ds(0, 112)], acc.at[pl.ds(t0 + 2 * CHUNK, 112)])

    @pl.when(s == NSUB - 1)
    def _():
        pltpu.sync_copy(rows.at[pl.ds(0, 16)], acc.at[pl.ds(9984, 16)])

    plsc.subcore_barrier()

    def sup_body(u, _):
        r0 = s * (EPT // 128) + u * 8
        pltpu.sync_copy(src2d.at[pl.ds(r0, 8)], sidx)
        pltpu.sync_copy(dst2d.at[pl.ds(r0, 8)], didx)
        pltpu.sync_copy(wflat.at[pl.ds(s * EPT + u * 1024, 1024)], wbuf)
        sidx[...] = sidx[...] + cN

        def chunk_body(q, _):
            cpa = pltpu.async_copy(y.at[sidx.at[2 * q]],
                                   rows.at[pl.ds(0, 128)], sem)
            cpb = pltpu.async_copy(y.at[sidx.at[2 * q + 1]],
                                   rows.at[pl.ds(128, 128)], sem)
            cpa.wait()
            cpb.wait()

            def scale(e, _):
                wv1 = wbuf[pl.ds(q * CHUNK + e, 1)]
                rows[e, :] = rows[e, :] * wv1[0]
                return _
            lax.fori_loop(0, CHUNK, scale, 0)

            cpc = pltpu.async_copy(rows.at[pl.ds(0, 128)],
                                   acc.at[didx.at[2 * q]], sem2, add=True)
            cpd = pltpu.async_copy(rows.at[pl.ds(128, 128)],
                                   acc.at[didx.at[2 * q + 1]], sem2, add=True)
            cpc.wait()
            cpd.wait()
            return _

        lax.fori_loop(0, 1024 // CHUNK, chunk_body, 0)
        return _

    lax.fori_loop(0, EPT // 1024, sup_body, 0)

    plsc.subcore_barrier()

    pltpu.sync_copy(acc.at[pl.ds(t0, 624)], agg.at[pl.ds(cN + t0, 624)])

    @pl.when(s == NSUB - 1)
    def _():
        pltpu.sync_copy(acc.at[pl.ds(9984, 16)], agg.at[pl.ds(cN + 9984, 16)])


def _edge_agg(y, src2d, dst2d, wflat):
    return pl.kernel(
        _agg_body,
        out_type=jax.ShapeDtypeStruct((NCORE * N, HD), jnp.float32),
        mesh=_mesh(),
        scratch_types=[
            pltpu.VMEM((8, 128), jnp.int32),
            pltpu.VMEM((8, 128), jnp.int32),
            pltpu.VMEM((1024,), jnp.float32),
            pltpu.VMEM((CHUNK, HD), jnp.float32),
            pltpu.VMEM_SHARED((N, HD), jnp.float32),
            pltpu.SemaphoreType.DMA,
            pltpu.SemaphoreType.DMA,
        ],
    )(y, src2d, dst2d, wflat)


def _dis_body(degp, dis):
    deg = degp[0] + degp[1] + 1.0
    dis[...] = lax.rsqrt(deg)[:, None]


def _dis_from_partials(degp):
    return pl.pallas_call(
        _dis_body,
        in_specs=[pl.BlockSpec((2, N), lambda: (0, 0))],
        out_specs=pl.BlockSpec((N, 1), lambda: (0, 0)),
        out_shape=jax.ShapeDtypeStruct((N, 1), jnp.float32),
    )(degp)


def _k0_body(x, wemb, bemb, w1, disr, y1):
    dis = disr[...]
    h = jnp.dot(x[...], wemb[...], preferred_element_type=jnp.float32)
    h = h + bemb[...]
    y1[...] = dis * jnp.dot(h, w1[...], preferred_element_type=jnp.float32)


def _embed_layer1(x, wemb, bemb, w1, disr):
    return pl.pallas_call(
        _k0_body,
        grid=(NB, 2),
        in_specs=[
            pl.BlockSpec((RB, D), lambda i, h: (i, 0)),
            pl.BlockSpec((D, D), lambda i, h: (0, 0)),
            pl.BlockSpec((1, D), lambda i, h: (0, 0)),
            pl.BlockSpec((D, HD), lambda i, h: (0, h)),
            pl.BlockSpec((RB, 1), lambda i, h: (i, 0)),
        ],
        out_specs=pl.BlockSpec((RB, HD), lambda i, h: (h * NB + i, 0)),
        out_shape=jax.ShapeDtypeStruct((NCORE * N, HD), jnp.float32),
    )(x, wemb, bemb, w1, disr)


def _kmid_body(aggA, aggB, yA, yB, b, wnext, disr, ynext):
    dis = disr[...]
    hfull = jnp.concatenate([aggA[...] + yA[...], aggB[...] + yB[...]], axis=1)
    hl = jnp.maximum(dis * hfull + b[...], 0.0)
    ynext[...] = dis * jnp.dot(hl, wnext[...],
                               preferred_element_type=jnp.float32)


def _mid_layer(agg, y, b, wnext, disr):
    return pl.pallas_call(
        _kmid_body,
        grid=(NB, 2),
        in_specs=[
            pl.BlockSpec((RB, HD), lambda i, h: (i, 0)),
            pl.BlockSpec((RB, HD), lambda i, h: (NB + i, 0)),
            pl.BlockSpec((RB, HD), lambda i, h: (i, 0)),
            pl.BlockSpec((RB, HD), lambda i, h: (NB + i, 0)),
            pl.BlockSpec((1, D), lambda i, h: (0, 0)),
            pl.BlockSpec((D, HD), lambda i, h: (0, h)),
            pl.BlockSpec((RB, 1), lambda i, h: (i, 0)),
        ],
        out_specs=pl.BlockSpec((RB, HD), lambda i, h: (h * NB + i, 0)),
        out_shape=jax.ShapeDtypeStruct((NCORE * N, HD), jnp.float32),
    )(agg, agg, y, y, b, wnext, disr)


def _klast_body(aggA, aggB, yA, yB, b4, wr1, br1, wr2, br2, wr3, br3, disr,
                out, accs):
    i = pl.program_id(0)

    @pl.when(i == 0)
    def _():
        accs[...] = jnp.zeros_like(accs)

    dis = disr[...]
    hfull = jnp.concatenate([aggA[...] + yA[...], aggB[...] + yB[...]], axis=1)
    hl = jnp.maximum(dis * hfull + b4[...], 0.0)
    accs[...] += jnp.sum(hl, axis=0, keepdims=True)

    @pl.when(i == NB - 1)
    def _():
        g = accs[...] / float(N)
        g8 = jnp.broadcast_to(g, (8, D))
        r = jnp.maximum(
            jnp.dot(g8, wr1[...], preferred_element_type=jnp.float32)
            + br1[...], 0.0)
        r = jnp.maximum(
            jnp.dot(r, wr2[...], preferred_element_type=jnp.float32)
            + br2[...], 0.0)
        o = jnp.dot(r, wr3[...], preferred_element_type=jnp.float32) + br3[...]
        out[...] = o[0:1, :]


def _pool_head(agg, y, b4, wr1, br1, wr2, br2, wr3, br3, disr):
    return pl.pallas_call(
        _klast_body,
        grid=(NB,),
        in_specs=[
            pl.BlockSpec((RB, HD), lambda i: (i, 0)),
            pl.BlockSpec((RB, HD), lambda i: (NB + i, 0)),
            pl.BlockSpec((RB, HD), lambda i: (i, 0)),
            pl.BlockSpec((RB, HD), lambda i: (NB + i, 0)),
            pl.BlockSpec((1, D), lambda i: (0, 0)),
            pl.BlockSpec((D, HD), lambda i: (0, 0)),
            pl.BlockSpec((1, HD), lambda i: (0, 0)),
            pl.BlockSpec((HD, 64), lambda i: (0, 0)),
            pl.BlockSpec((1, 64), lambda i: (0, 0)),
            pl.BlockSpec((64, 10), lambda i: (0, 0)),
            pl.BlockSpec((1, 10), lambda i: (0, 0)),
            pl.BlockSpec((RB, 1), lambda i: (i, 0)),
        ],
        out_specs=pl.BlockSpec((1, 10), lambda i: (0, 0)),
        out_shape=jax.ShapeDtypeStruct((1, 10), jnp.float32),
        scratch_shapes=[pltpu.VMEM((1, D), jnp.float32)],
    )(agg, agg, y, y, b4, wr1, br1, wr2, br2, wr3, br3, disr)


def kernel(x, edge_index, edge_weight, W_emb, b_emb, W1, b1, W2, b2, W3, b3,
           W4, b4, Wr1, br1, Wr2, br2, Wr3, br3):
    src = edge_index[0]
    dst = edge_index[1]
    pad = E_PAD - E
    zi = jnp.zeros((pad,), jnp.int32)
    src2d = jnp.concatenate([src, zi]).reshape(ROWS, 128)
    dst2d = jnp.concatenate([dst, zi]).reshape(ROWS, 128)
    wflat = jnp.concatenate([edge_weight, jnp.zeros((pad,), jnp.float32)])

    degp = _deg_partials(dst2d, wflat).reshape(NCORE, N)
    disr = _dis_from_partials(degp)

    bemb = b_emb.reshape(1, D)
    y = _embed_layer1(x, W_emb, bemb, W1, disr)
    for b_l, w_next in ((b1, W2), (b2, W3), (b3, W4)):
        agg = _edge_agg(y, src2d, dst2d, wflat)
        y = _mid_layer(agg, y, b_l.reshape(1, D), w_next, disr)
    agg = _edge_agg(y, src2d, dst2d, wflat)
    return _pool_head(agg, y, b4.reshape(1, D), Wr1, br1.reshape(1, HD),
                      Wr2, br2.reshape(1, 64), Wr3, br3.reshape(1, 10), disr)


# speedup vs baseline: 5.4046x; 1.1405x over previous
import functools

import jax
import jax.numpy as jnp
from jax import lax
from jax.experimental import pallas as pl
from jax.experimental.pallas import tpu as pltpu
from jax.experimental.pallas import tpu_sc as plsc

N = 10000
E = 160000
D = 256
HD = 128
NCORE = 2
NSUB = 16
LANES = 16
EPT = 10240
E_PAD = EPT * NSUB
ROWS = E_PAD // 128
CHUNK = 256
NCHUNK = EPT // CHUNK
RB = 1000
NB = N // RB


def _mesh():
    return plsc.VectorSubcoreMesh(core_axis_name="c", subcore_axis_name="s")


def _deg_body(dst2d, wflat, degp, dbuf, wbuf, zbuf, acc, sem):
    c = lax.axis_index("c")
    s = lax.axis_index("s")
    wid = c * NSUB + s
    row0 = wid * (ROWS // (NCORE * NSUB))
    ebase = wid * (E_PAD // (NCORE * NSUB))

    @pl.when(s == 0)
    def _():
        zbuf[...] = jnp.zeros((1024,), jnp.float32)
        for k in range(9):
            pltpu.sync_copy(zbuf, acc.at[pl.ds(k * 1024, 1024)])
        pltpu.sync_copy(zbuf.at[pl.ds(0, 784)], acc.at[pl.ds(9216, 784)])

    plsc.subcore_barrier()

    pltpu.sync_copy(dst2d.at[pl.ds(row0, 40)], dbuf)
    pltpu.sync_copy(wflat.at[pl.ds(ebase, 5120)], wbuf)
    cps = [
        pltpu.async_copy(wbuf.at[pl.ds(r * 128, 128)], acc.at[dbuf.at[r]],
                         sem, add=True)
        for r in range(40)
    ]
    for cp in cps:
        cp.wait()

    plsc.subcore_barrier()

    @pl.when(s == 0)
    def _():
        pltpu.sync_copy(acc.at[pl.ds(0, 5120)], wbuf)
        pltpu.sync_copy(wbuf, degp.at[pl.ds(c * N, 5120)])
        pltpu.sync_copy(acc.at[pl.ds(5120, 4880)], wbuf.at[pl.ds(0, 4880)])
        pltpu.sync_copy(wbuf.at[pl.ds(0, 4880)],
                        degp.at[pl.ds(c * N + 5120, 4880)])


def _deg_partials(dst2d, wflat):
    return pl.kernel(
        _deg_body,
        out_type=jax.ShapeDtypeStruct((NCORE * N,), jnp.float32),
        mesh=_mesh(),
        scratch_types=[
            pltpu.VMEM((40, 128), jnp.int32),
            pltpu.VMEM((5120,), jnp.float32),
            pltpu.VMEM((1024,), jnp.float32),
            pltpu.VMEM_SHARED((N,), jnp.float32),
            pltpu.SemaphoreType.DMA,
        ],
    )(dst2d, wflat)


def _agg_body(y, src2d, dst2d, wflat, agg, sidx, didx, wbuf, rows, acc,
              sem, sem2):
    c = lax.axis_index("c")
    s = lax.axis_index("s")
    cN = c * N

    rows[...] = jnp.zeros((CHUNK, HD), jnp.float32)
    t0 = s * 624
    pltpu.sync_copy(rows, acc.at[pl.ds(t0, CHUNK)])
    pltpu.sync_copy(rows, acc.at[pl.ds(t0 + CHUNK, CHUNK)])
    pltpu.sync_copy(rows.at[pl.ds(0, 112)], acc.at[pl.ds(t0 + 2 * CHUNK, 112)])

    @pl.when(s == NSUB - 1)
    def _():
        pltpu.sync_copy(rows.at[pl.ds(0, 16)], acc.at[pl.ds(9984, 16)])

    plsc.subcore_barrier()

    def sup_body(u, _):
        r0 = s * (EPT // 128) + u * 8
        pltpu.sync_copy(src2d.at[pl.ds(r0, 8)], sidx)
        pltpu.sync_copy(dst2d.at[pl.ds(r0, 8)], didx)
        pltpu.sync_copy(wflat.at[pl.ds(s * EPT + u * 1024, 1024)], wbuf)
        sidx[...] = sidx[...] + cN

        def chunk_body(q, _):
            cpa = pltpu.async_copy(y.at[sidx.at[2 * q]],
                                   rows.at[pl.ds(0, 128)], sem)
            cpb = pltpu.async_copy(y.at[sidx.at[2 * q + 1]],
                                   rows.at[pl.ds(128, 128)], sem)
            cpa.wait()
            cpb.wait()

            @plsc.parallel_loop(0, CHUNK, unroll=8)
            def _scale(e):
                wv1 = wbuf[pl.ds(q * CHUNK + e, 1)]
                rows[e, :] = rows[e, :] * wv1[0]

            cpc = pltpu.async_copy(rows.at[pl.ds(0, 128)],
                                   acc.at[didx.at[2 * q]], sem2, add=True)
            cpd = pltpu.async_copy(rows.at[pl.ds(128, 128)],
                                   acc.at[didx.at[2 * q + 1]], sem2, add=True)
            cpc.wait()
            cpd.wait()
            return _

        lax.fori_loop(0, 1024 // CHUNK, chunk_body, 0)
        return _

    lax.fori_loop(0, EPT // 1024, sup_body, 0)

    plsc.subcore_barrier()

    pltpu.sync_copy(acc.at[pl.ds(t0, 624)], agg.at[pl.ds(cN + t0, 624)])

    @pl.when(s == NSUB - 1)
    def _():
        pltpu.sync_copy(acc.at[pl.ds(9984, 16)], agg.at[pl.ds(cN + 9984, 16)])


def _edge_agg(y, src2d, dst2d, wflat):
    return pl.kernel(
        _agg_body,
        out_type=jax.ShapeDtypeStruct((NCORE * N, HD), jnp.float32),
        mesh=_mesh(),
        scratch_types=[
            pltpu.VMEM((8, 128), jnp.int32),
            pltpu.VMEM((8, 128), jnp.int32),
            pltpu.VMEM((1024,), jnp.float32),
            pltpu.VMEM((CHUNK, HD), jnp.float32),
            pltpu.VMEM_SHARED((N, HD), jnp.float32),
            pltpu.SemaphoreType.DMA,
            pltpu.SemaphoreType.DMA,
        ],
    )(y, src2d, dst2d, wflat)


def _dis_body(degp, dis):
    deg = degp[0] + degp[1] + 1.0
    dis[...] = lax.rsqrt(deg)[:, None]


def _dis_from_partials(degp):
    return pl.pallas_call(
        _dis_body,
        in_specs=[pl.BlockSpec((2, N), lambda: (0, 0))],
        out_specs=pl.BlockSpec((N, 1), lambda: (0, 0)),
        out_shape=jax.ShapeDtypeStruct((N, 1), jnp.float32),
    )(degp)


def _k0_body(x, wemb, bemb, w1, disr, y1):
    dis = disr[...]
    h = jnp.dot(x[...], wemb[...], preferred_element_type=jnp.float32)
    h = h + bemb[...]
    y1[...] = dis * jnp.dot(h, w1[...], preferred_element_type=jnp.float32)


def _embed_layer1(x, wemb, bemb, w1, disr):
    return pl.pallas_call(
        _k0_body,
        grid=(NB, 2),
        in_specs=[
            pl.BlockSpec((RB, D), lambda i, h: (i, 0)),
            pl.BlockSpec((D, D), lambda i, h: (0, 0)),
            pl.BlockSpec((1, D), lambda i, h: (0, 0)),
            pl.BlockSpec((D, HD), lambda i, h: (0, h)),
            pl.BlockSpec((RB, 1), lambda i, h: (i, 0)),
        ],
        out_specs=pl.BlockSpec((RB, HD), lambda i, h: (h * NB + i, 0)),
        out_shape=jax.ShapeDtypeStruct((NCORE * N, HD), jnp.float32),
    )(x, wemb, bemb, w1, disr)


def _kmid_body(aggA, aggB, yA, yB, b, wnext, disr, ynext):
    dis = disr[...]
    hfull = jnp.concatenate([aggA[...] + yA[...], aggB[...] + yB[...]], axis=1)
    hl = jnp.maximum(dis * hfull + b[...], 0.0)
    ynext[...] = dis * jnp.dot(hl, wnext[...],
                               preferred_element_type=jnp.float32)


def _mid_layer(agg, y, b, wnext, disr):
    return pl.pallas_call(
        _kmid_body,
        grid=(NB, 2),
        in_specs=[
            pl.BlockSpec((RB, HD), lambda i, h: (i, 0)),
            pl.BlockSpec((RB, HD), lambda i, h: (NB + i, 0)),
            pl.BlockSpec((RB, HD), lambda i, h: (i, 0)),
            pl.BlockSpec((RB, HD), lambda i, h: (NB + i, 0)),
            pl.BlockSpec((1, D), lambda i, h: (0, 0)),
            pl.BlockSpec((D, HD), lambda i, h: (0, h)),
            pl.BlockSpec((RB, 1), lambda i, h: (i, 0)),
        ],
        out_specs=pl.BlockSpec((RB, HD), lambda i, h: (h * NB + i, 0)),
        out_shape=jax.ShapeDtypeStruct((NCORE * N, HD), jnp.float32),
    )(agg, agg, y, y, b, wnext, disr)


def _klast_body(aggA, aggB, yA, yB, b4, wr1, br1, wr2, br2, wr3, br3, disr,
                out, accs):
    i = pl.program_id(0)

    @pl.when(i == 0)
    def _():
        accs[...] = jnp.zeros_like(accs)

    dis = disr[...]
    hfull = jnp.concatenate([aggA[...] + yA[...], aggB[...] + yB[...]], axis=1)
    hl = jnp.maximum(dis * hfull + b4[...], 0.0)
    accs[...] += jnp.sum(hl, axis=0, keepdims=True)

    @pl.when(i == NB - 1)
    def _():
        g = accs[...] / float(N)
        g8 = jnp.broadcast_to(g, (8, D))
        r = jnp.maximum(
            jnp.dot(g8, wr1[...], preferred_element_type=jnp.float32)
            + br1[...], 0.0)
        r = jnp.maximum(
            jnp.dot(r, wr2[...], preferred_element_type=jnp.float32)
            + br2[...], 0.0)
        o = jnp.dot(r, wr3[...], preferred_element_type=jnp.float32) + br3[...]
        out[...] = o[0:1, :]


def _pool_head(agg, y, b4, wr1, br1, wr2, br2, wr3, br3, disr):
    return pl.pallas_call(
        _klast_body,
        grid=(NB,),
        in_specs=[
            pl.BlockSpec((RB, HD), lambda i: (i, 0)),
            pl.BlockSpec((RB, HD), lambda i: (NB + i, 0)),
            pl.BlockSpec((RB, HD), lambda i: (i, 0)),
            pl.BlockSpec((RB, HD), lambda i: (NB + i, 0)),
            pl.BlockSpec((1, D), lambda i: (0, 0)),
            pl.BlockSpec((D, HD), lambda i: (0, 0)),
            pl.BlockSpec((1, HD), lambda i: (0, 0)),
            pl.BlockSpec((HD, 64), lambda i: (0, 0)),
            pl.BlockSpec((1, 64), lambda i: (0, 0)),
            pl.BlockSpec((64, 10), lambda i: (0, 0)),
            pl.BlockSpec((1, 10), lambda i: (0, 0)),
            pl.BlockSpec((RB, 1), lambda i: (i, 0)),
        ],
        out_specs=pl.BlockSpec((1, 10), lambda i: (0, 0)),
        out_shape=jax.ShapeDtypeStruct((1, 10), jnp.float32),
        scratch_shapes=[pltpu.VMEM((1, D), jnp.float32)],
    )(agg, agg, y, y, b4, wr1, br1, wr2, br2, wr3, br3, disr)


def kernel(x, edge_index, edge_weight, W_emb, b_emb, W1, b1, W2, b2, W3, b3,
           W4, b4, Wr1, br1, Wr2, br2, Wr3, br3):
    src = edge_index[0]
    dst = edge_index[1]
    pad = E_PAD - E
    zi = jnp.zeros((pad,), jnp.int32)
    src2d = jnp.concatenate([src, zi]).reshape(ROWS, 128)
    dst2d = jnp.concatenate([dst, zi]).reshape(ROWS, 128)
    wflat = jnp.concatenate([edge_weight, jnp.zeros((pad,), jnp.float32)])

    degp = _deg_partials(dst2d, wflat).reshape(NCORE, N)
    disr = _dis_from_partials(degp)

    bemb = b_emb.reshape(1, D)
    y = _embed_layer1(x, W_emb, bemb, W1, disr)
    for b_l, w_next in ((b1, W2), (b2, W3), (b3, W4)):
        agg = _edge_agg(y, src2d, dst2d, wflat)
        y = _mid_layer(agg, y, b_l.reshape(1, D), w_next, disr)
    agg = _edge_agg(y, src2d, dst2d, wflat)
    return _pool_head(agg, y, b4.reshape(1, D), Wr1, br1.reshape(1, HD),
                      Wr2, br2.reshape(1, 64), Wr3, br3.reshape(1, 10), disr)


# speedup vs baseline: 6.4141x; 1.1868x over previous
import functools

import jax
import jax.numpy as jnp
from jax import lax
from jax.experimental import pallas as pl
from jax.experimental.pallas import tpu as pltpu
from jax.experimental.pallas import tpu_sc as plsc

N = 10000
E = 160000
D = 256
HD = 128
NCORE = 2
NSUB = 16
LANES = 16
EPT = 10240
E_PAD = EPT * NSUB
ROWS = E_PAD // 128
CHUNK = 128
NCHUNK = EPT // CHUNK
RB = 1000
NB = N // RB


def _mesh():
    return plsc.VectorSubcoreMesh(core_axis_name="c", subcore_axis_name="s")


def _deg_body(dst2d, wflat, degp, dbuf, wbuf, zbuf, acc, sem):
    c = lax.axis_index("c")
    s = lax.axis_index("s")
    wid = c * NSUB + s
    row0 = wid * (ROWS // (NCORE * NSUB))
    ebase = wid * (E_PAD // (NCORE * NSUB))

    @pl.when(s == 0)
    def _():
        zbuf[...] = jnp.zeros((1024,), jnp.float32)
        for k in range(9):
            pltpu.sync_copy(zbuf, acc.at[pl.ds(k * 1024, 1024)])
        pltpu.sync_copy(zbuf.at[pl.ds(0, 784)], acc.at[pl.ds(9216, 784)])

    plsc.subcore_barrier()

    pltpu.sync_copy(dst2d.at[pl.ds(row0, 40)], dbuf)
    pltpu.sync_copy(wflat.at[pl.ds(ebase, 5120)], wbuf)
    cps = [
        pltpu.async_copy(wbuf.at[pl.ds(r * 128, 128)], acc.at[dbuf.at[r]],
                         sem, add=True)
        for r in range(40)
    ]
    for cp in cps:
        cp.wait()

    plsc.subcore_barrier()

    @pl.when(s == 0)
    def _():
        pltpu.sync_copy(acc.at[pl.ds(0, 5120)], wbuf)
        pltpu.sync_copy(wbuf, degp.at[pl.ds(c * N, 5120)])
        pltpu.sync_copy(acc.at[pl.ds(5120, 4880)], wbuf.at[pl.ds(0, 4880)])
        pltpu.sync_copy(wbuf.at[pl.ds(0, 4880)],
                        degp.at[pl.ds(c * N + 5120, 4880)])


def _deg_partials(dst2d, wflat):
    return pl.kernel(
        _deg_body,
        out_type=jax.ShapeDtypeStruct((NCORE * N,), jnp.float32),
        mesh=_mesh(),
        scratch_types=[
            pltpu.VMEM((40, 128), jnp.int32),
            pltpu.VMEM((5120,), jnp.float32),
            pltpu.VMEM((1024,), jnp.float32),
            pltpu.VMEM_SHARED((N,), jnp.float32),
            pltpu.SemaphoreType.DMA,
        ],
    )(dst2d, wflat)


def _agg_body(y, src2d, dst2d, wflat, agg,
              sidx0, sidx1, didx0, didx1, wbuf, rows0, rows1, acc,
              gsem0, gsem1, ssem0, ssem1, isem):
    c = lax.axis_index("c")
    s = lax.axis_index("s")
    cN = c * N
    sidx = (sidx0, sidx1)
    didx = (didx0, didx1)
    rows = (rows0, rows1)
    gsem = (gsem0, gsem1)
    ssem = (ssem0, ssem1)

    rows0[...] = jnp.zeros((CHUNK, HD), jnp.float32)
    rows1[...] = jnp.zeros((CHUNK, HD), jnp.float32)
    t0 = s * 624
    for k in range(4):
        pltpu.sync_copy(rows0, acc.at[pl.ds(t0 + k * CHUNK, CHUNK)])
    pltpu.sync_copy(rows0.at[pl.ds(0, 112)], acc.at[pl.ds(t0 + 4 * CHUNK, 112)])

    @pl.when(s == NSUB - 1)
    def _():
        pltpu.sync_copy(rows0.at[pl.ds(0, 16)], acc.at[pl.ds(9984, 16)])

    pltpu.sync_copy(wflat.at[pl.ds(s * EPT, EPT)], wbuf)
    pltpu.sync_copy(src2d.at[pl.ds(s * (EPT // 128), 8)], sidx0)
    pltpu.sync_copy(dst2d.at[pl.ds(s * (EPT // 128), 8)], didx0)
    sidx0[...] = sidx0[...] + cN

    plsc.subcore_barrier()

    pltpu.async_copy(rows1, acc.at[didx0.at[0]], ssem1, add=True)
    pltpu.async_copy(y.at[sidx0.at[0]], rows0, gsem0)

    def sup(k, u_par, base_q):
        sl = sidx[u_par]
        dl = didx[u_par]
        nsl = sidx[1 - u_par]
        ndl = didx[1 - u_par]
        pltpu.make_async_copy(rows1, acc.at[dl.at[0]], ssem1).wait()
        u = 2 * k + u_par
        nrow = jnp.minimum(u + 1, EPT // 1024 - 1) * 8 + s * (EPT // 128)
        cp_si = pltpu.async_copy(src2d.at[pl.ds(nrow, 8)], nsl, isem)
        cp_di = pltpu.async_copy(dst2d.at[pl.ds(nrow, 8)], ndl, isem)
        for q in range(8):
            p = q & 1
            Q = base_q + q
            pltpu.make_async_copy(y.at[sl.at[q]], rows[p], gsem[p]).wait()
            if q >= 1:
                pltpu.make_async_copy(rows[1 - p], acc.at[dl.at[0]],
                                      ssem[1 - p]).wait()
            if q < 7:
                pltpu.async_copy(y.at[sl.at[q + 1]], rows[1 - p],
                                 gsem[1 - p])
            else:
                cp_si.wait()
                cp_di.wait()
                nsl[...] = nsl[...] + cN
                pltpu.async_copy(y.at[nsl.at[0]], rows[1 - p], gsem[1 - p])

            @plsc.parallel_loop(0, CHUNK, unroll=8)
            def _scale(e):
                wv1 = wbuf[pl.ds(Q * CHUNK + e, 1)]
                rows[p][e, :] = rows[p][e, :] * wv1[0]

            pltpu.async_copy(rows[p], acc.at[dl.at[q]], ssem[p], add=True)

    def pair_body(k, _):
        sup(k, 0, 16 * k)
        sup(k, 1, 16 * k + 8)
        return _

    lax.fori_loop(0, EPT // 2048, pair_body, 0)

    pltpu.make_async_copy(y.at[sidx0.at[0]], rows0, gsem0).wait()
    pltpu.make_async_copy(rows1, acc.at[didx0.at[0]], ssem1).wait()

    plsc.subcore_barrier()

    pltpu.sync_copy(acc.at[pl.ds(t0, 624)], agg.at[pl.ds(cN + t0, 624)])

    @pl.when(s == NSUB - 1)
    def _():
        pltpu.sync_copy(acc.at[pl.ds(9984, 16)], agg.at[pl.ds(cN + 9984, 16)])


def _edge_agg(y, src2d, dst2d, wflat):
    return pl.kernel(
        _agg_body,
        out_type=jax.ShapeDtypeStruct((NCORE * N, HD), jnp.float32),
        mesh=_mesh(),
        scratch_types=[
            pltpu.VMEM((8, 128), jnp.int32),
            pltpu.VMEM((8, 128), jnp.int32),
            pltpu.VMEM((8, 128), jnp.int32),
            pltpu.VMEM((8, 128), jnp.int32),
            pltpu.VMEM((EPT,), jnp.float32),
            pltpu.VMEM((CHUNK, HD), jnp.float32),
            pltpu.VMEM((CHUNK, HD), jnp.float32),
            pltpu.VMEM_SHARED((N, HD), jnp.float32),
            pltpu.SemaphoreType.DMA,
            pltpu.SemaphoreType.DMA,
            pltpu.SemaphoreType.DMA,
            pltpu.SemaphoreType.DMA,
            pltpu.SemaphoreType.DMA,
        ],
    )(y, src2d, dst2d, wflat)


def _dis_body(degp, dis):
    deg = degp[0] + degp[1] + 1.0
    dis[...] = lax.rsqrt(deg)[:, None]


def _dis_from_partials(degp):
    return pl.pallas_call(
        _dis_body,
        in_specs=[pl.BlockSpec((2, N), lambda: (0, 0))],
        out_specs=pl.BlockSpec((N, 1), lambda: (0, 0)),
        out_shape=jax.ShapeDtypeStruct((N, 1), jnp.float32),
    )(degp)


def _k0_body(x, wemb, bemb, w1, disr, y1):
    dis = disr[...]
    h = jnp.dot(x[...], wemb[...], preferred_element_type=jnp.float32)
    h = h + bemb[...]
    y1[...] = dis * jnp.dot(h, w1[...], preferred_element_type=jnp.float32)


def _embed_layer1(x, wemb, bemb, w1, disr):
    return pl.pallas_call(
        _k0_body,
        grid=(NB, 2),
        in_specs=[
            pl.BlockSpec((RB, D), lambda i, h: (i, 0)),
            pl.BlockSpec((D, D), lambda i, h: (0, 0)),
            pl.BlockSpec((1, D), lambda i, h: (0, 0)),
            pl.BlockSpec((D, HD), lambda i, h: (0, h)),
            pl.BlockSpec((RB, 1), lambda i, h: (i, 0)),
        ],
        out_specs=pl.BlockSpec((RB, HD), lambda i, h: (h * NB + i, 0)),
        out_shape=jax.ShapeDtypeStruct((NCORE * N, HD), jnp.float32),
    )(x, wemb, bemb, w1, disr)


def _kmid_body(aggA, aggB, yA, yB, b, wnext, disr, ynext):
    dis = disr[...]
    hfull = jnp.concatenate([aggA[...] + yA[...], aggB[...] + yB[...]], axis=1)
    hl = jnp.maximum(dis * hfull + b[...], 0.0)
    ynext[...] = dis * jnp.dot(hl, wnext[...],
                               preferred_element_type=jnp.float32)


def _mid_layer(agg, y, b, wnext, disr):
    return pl.pallas_call(
        _kmid_body,
        grid=(NB, 2),
        in_specs=[
            pl.BlockSpec((RB, HD), lambda i, h: (i, 0)),
            pl.BlockSpec((RB, HD), lambda i, h: (NB + i, 0)),
            pl.BlockSpec((RB, HD), lambda i, h: (i, 0)),
            pl.BlockSpec((RB, HD), lambda i, h: (NB + i, 0)),
            pl.BlockSpec((1, D), lambda i, h: (0, 0)),
            pl.BlockSpec((D, HD), lambda i, h: (0, h)),
            pl.BlockSpec((RB, 1), lambda i, h: (i, 0)),
        ],
        out_specs=pl.BlockSpec((RB, HD), lambda i, h: (h * NB + i, 0)),
        out_shape=jax.ShapeDtypeStruct((NCORE * N, HD), jnp.float32),
    )(agg, agg, y, y, b, wnext, disr)


def _klast_body(aggA, aggB, yA, yB, b4, wr1, br1, wr2, br2, wr3, br3, disr,
                out, accs):
    i = pl.program_id(0)

    @pl.when(i == 0)
    def _():
        accs[...] = jnp.zeros_like(accs)

    dis = disr[...]
    hfull = jnp.concatenate([aggA[...] + yA[...], aggB[...] + yB[...]], axis=1)
    hl = jnp.maximum(dis * hfull + b4[...], 0.0)
    accs[...] += jnp.sum(hl, axis=0, keepdims=True)

    @pl.when(i == NB - 1)
    def _():
        g = accs[...] / float(N)
        g8 = jnp.broadcast_to(g, (8, D))
        r = jnp.maximum(
            jnp.dot(g8, wr1[...], preferred_element_type=jnp.float32)
            + br1[...], 0.0)
        r = jnp.maximum(
            jnp.dot(r, wr2[...], preferred_element_type=jnp.float32)
            + br2[...], 0.0)
        o = jnp.dot(r, wr3[...], preferred_element_type=jnp.float32) + br3[...]
        out[...] = o[0:1, :]


def _pool_head(agg, y, b4, wr1, br1, wr2, br2, wr3, br3, disr):
    return pl.pallas_call(
        _klast_body,
        grid=(NB,),
        in_specs=[
            pl.BlockSpec((RB, HD), lambda i: (i, 0)),
            pl.BlockSpec((RB, HD), lambda i: (NB + i, 0)),
            pl.BlockSpec((RB, HD), lambda i: (i, 0)),
            pl.BlockSpec((RB, HD), lambda i: (NB + i, 0)),
            pl.BlockSpec((1, D), lambda i: (0, 0)),
            pl.BlockSpec((D, HD), lambda i: (0, 0)),
            pl.BlockSpec((1, HD), lambda i: (0, 0)),
            pl.BlockSpec((HD, 64), lambda i: (0, 0)),
            pl.BlockSpec((1, 64), lambda i: (0, 0)),
            pl.BlockSpec((64, 10), lambda i: (0, 0)),
            pl.BlockSpec((1, 10), lambda i: (0, 0)),
            pl.BlockSpec((RB, 1), lambda i: (i, 0)),
        ],
        out_specs=pl.BlockSpec((1, 10), lambda i: (0, 0)),
        out_shape=jax.ShapeDtypeStruct((1, 10), jnp.float32),
        scratch_shapes=[pltpu.VMEM((1, D), jnp.float32)],
    )(agg, agg, y, y, b4, wr1, br1, wr2, br2, wr3, br3, disr)


def kernel(x, edge_index, edge_weight, W_emb, b_emb, W1, b1, W2, b2, W3, b3,
           W4, b4, Wr1, br1, Wr2, br2, Wr3, br3):
    src = edge_index[0]
    dst = edge_index[1]
    pad = E_PAD - E
    zi = jnp.zeros((pad,), jnp.int32)
    src2d = jnp.concatenate([src, zi]).reshape(ROWS, 128)
    dst2d = jnp.concatenate([dst, zi]).reshape(ROWS, 128)
    wflat = jnp.concatenate([edge_weight, jnp.zeros((pad,), jnp.float32)])

    degp = _deg_partials(dst2d, wflat).reshape(NCORE, N)
    disr = _dis_from_partials(degp)

    bemb = b_emb.reshape(1, D)
    y = _embed_layer1(x, W_emb, bemb, W1, disr)
    for b_l, w_next in ((b1, W2), (b2, W3), (b3, W4)):
        agg = _edge_agg(y, src2d, dst2d, wflat)
        y = _mid_layer(agg, y, b_l.reshape(1, D), w_next, disr)
    agg = _edge_agg(y, src2d, dst2d, wflat)
    return _pool_head(agg, y, b4.reshape(1, D), Wr1, br1.reshape(1, HD),
                      Wr2, br2.reshape(1, 64), Wr3, br3.reshape(1, 10), disr)
